# baseline (speedup 1.0000x reference)
import functools

import jax
import jax.numpy as jnp
from jax import lax
from jax.experimental import pallas as pl
from jax.experimental.pallas import tpu as pltpu
from jax.experimental.pallas import tpu_sc as plsc

SEQ = 200
BATCH = 4096
D = 32
BPW = 128


def _make_kernel():
  info = plsc.get_sparse_core_info()
  nc, ns = info.num_cores, info.num_subcores
  nw = nc * ns
  assert BPW * nw == BATCH
  n2 = SEQ // 2

  mesh = plsc.VectorSubcoreMesh(core_axis_name="c", subcore_axis_name="s")

  @functools.partial(
      pl.kernel,
      mesh=mesh,
      compiler_params=pltpu.CompilerParams(use_tc_tiling_on_sc=False,
                                           needs_layout_passes=False),
      out_type=jax.ShapeDtypeStruct((SEQ, D // 8, BATCH // BPW, 8, BPW),
                                    jnp.float32),
      scratch_types=[
          pltpu.VMEM((SEQ, BPW), jnp.int32),
          pltpu.VMEM((BPW, D), jnp.float32),
          pltpu.VMEM((BPW, D), jnp.float32),
          pltpu.VMEM((D, BPW), jnp.float32),
          pltpu.VMEM((D, BPW), jnp.float32),
          pltpu.SemaphoreType.DMA,
          pltpu.SemaphoreType.DMA,
          pltpu.SemaphoreType.DMA,
          pltpu.SemaphoreType.DMA,
      ],
  )
  def gather_kernel(emb_hbm, xt_hbm, out_hbm, idx_v,
                    rows_v0, rows_v1, tiles_v0, tiles_v1,
                    sg0, sg1, sw0, sw1):
    wid = lax.axis_index("s") * nc + lax.axis_index("c")
    b0 = wid * BPW

    pltpu.sync_copy(xt_hbm.at[:, pl.ds(b0, BPW)], idx_v)

    iota = lax.iota(jnp.int32, 16)
    c31 = jnp.full((16,), 31, jnp.int32)
    c7 = jnp.full((16,), 7, jnp.int32)

    def fire_g(rows_v, sem, l):
      pltpu.async_copy(emb_hbm.at[idx_v.at[l]], rows_v, sem)

    def drain_g(rows_v, sem, l):
      pltpu.make_async_copy(emb_hbm.at[idx_v.at[l]], rows_v, sem).wait()

    def fire_w(tiles_v, sem, l):
      for dt in range(D // 8):
        pltpu.async_copy(tiles_v.at[pl.ds(8 * dt, 8)], out_hbm.at[l, dt, wid],
                         sem)

    def drain_w(tiles_v, sem, l):
      for dt in range(D // 8):
        pltpu.make_async_copy(tiles_v.at[pl.ds(8 * dt, 8)],
                              out_hbm.at[l, dt, wid], sem).wait()

    m_ks = [iota + jnp.full((16,), 16 * k, jnp.int32)
            for k in range(BPW // 16)]

    def transpose(rows_v, tiles_v):
      def cbody(c0, carry):
        d = lax.bitwise_and(
            iota + jnp.broadcast_to(c0, (16,)).astype(jnp.int32), c31)
        for k in range(BPW // 16):
          x16 = plsc.load_gather(rows_v, [m_ks[k], d])
          plsc.store_scatter(tiles_v, [d, m_ks[k]], x16)
        return carry
      lax.fori_loop(0, D, cbody, 0)

    fire_g(rows_v0, sg0, 0)
    fire_g(rows_v1, sg1, 1)

    def body(g, carry):
      l0 = 2 * g

      drain_g(rows_v0, sg0, l0)
      @pl.when(g > 0)
      def _():
        drain_w(tiles_v0, sw0, l0 - 2)
      transpose(rows_v0, tiles_v0)
      fire_w(tiles_v0, sw0, l0)
      @pl.when(g < n2 - 1)
      def _():
        fire_g(rows_v0, sg0, l0 + 2)

      drain_g(rows_v1, sg1, l0 + 1)
      @pl.when(g > 0)
      def _():
        drain_w(tiles_v1, sw1, l0 - 1)
      transpose(rows_v1, tiles_v1)
      fire_w(tiles_v1, sw1, l0 + 1)
      @pl.when(g < n2 - 1)
      def _():
        fire_g(rows_v1, sg1, l0 + 3)
      return carry

    lax.fori_loop(0, n2, body, 0)

    drain_w(tiles_v0, sw0, SEQ - 2)
    drain_w(tiles_v1, sw1, SEQ - 1)

  return gather_kernel


_gather = _make_kernel()


def kernel(x, hidden, emb):
  del hidden
  xt = jnp.transpose(x.astype(jnp.int32))
  out5 = _gather(emb, xt)
  out6 = jnp.transpose(out5, (0, 2, 4, 1, 3))
  return out6.reshape(SEQ, BATCH, D)


# speedup vs baseline: 1.0123x; 1.0123x over previous
import functools

import jax
import jax.numpy as jnp
from jax import lax
from jax.experimental import pallas as pl
from jax.experimental.pallas import tpu as pltpu
from jax.experimental.pallas import tpu_sc as plsc

SEQ = 200
BATCH = 4096
D = 32
BPW = 128


def _make_kernel():
  info = plsc.get_sparse_core_info()
  nc, ns = info.num_cores, info.num_subcores
  nw = nc * ns
  assert BPW * nw == BATCH
  n2 = SEQ // 2

  mesh = plsc.VectorSubcoreMesh(core_axis_name="c", subcore_axis_name="s")

  @functools.partial(
      pl.kernel,
      mesh=mesh,
      compiler_params=pltpu.CompilerParams(use_tc_tiling_on_sc=False,
                                           needs_layout_passes=False),
      out_type=jax.ShapeDtypeStruct((SEQ, D // 8, BATCH // BPW, 8, BPW),
                                    jnp.float32),
      scratch_types=[
          pltpu.VMEM((SEQ, BPW), jnp.int32),
          pltpu.VMEM((BPW, D), jnp.float32),
          pltpu.VMEM((BPW, D), jnp.float32),
          pltpu.VMEM((D, BPW), jnp.float32),
          pltpu.VMEM((D, BPW), jnp.float32),
          pltpu.SemaphoreType.DMA,
          pltpu.SemaphoreType.DMA,
          pltpu.SemaphoreType.DMA,
          pltpu.SemaphoreType.DMA,
      ],
  )
  def gather_kernel(emb_hbm, xt_hbm, out_hbm, idx_v,
                    rows_v0, rows_v1, tiles_v0, tiles_v1,
                    sg0, sg1, sw0, sw1):
    wid = lax.axis_index("s") * nc + lax.axis_index("c")
    b0 = wid * BPW

    pltpu.sync_copy(xt_hbm.at[:, pl.ds(b0, BPW)], idx_v)

    iota = lax.iota(jnp.int32, 16)
    c31 = jnp.full((16,), 31, jnp.int32)
    c7 = jnp.full((16,), 7, jnp.int32)

    def fire_g(rows_v, sem, l):
      pltpu.async_copy(emb_hbm.at[idx_v.at[l]], rows_v, sem)

    def drain_g(rows_v, sem, l):
      pltpu.make_async_copy(emb_hbm.at[idx_v.at[l]], rows_v, sem).wait()

    def fire_w(tiles_v, sem, l):
      for dt in range(D // 8):
        pltpu.async_copy(tiles_v.at[pl.ds(8 * dt, 8)], out_hbm.at[l, dt, wid],
                         sem)

    def drain_w(tiles_v, sem, l):
      for dt in range(D // 8):
        pltpu.make_async_copy(tiles_v.at[pl.ds(8 * dt, 8)],
                              out_hbm.at[l, dt, wid], sem).wait()

    m_ks = [iota + jnp.full((16,), 16 * k, jnp.int32)
            for k in range(BPW // 16)]

    def transpose(rows_v, tiles_v):
      def cbody(c4, carry):
        base = jnp.broadcast_to(4 * c4, (16,)).astype(jnp.int32) + iota
        for u in range(4):
          d = lax.bitwise_and(base + jnp.full((16,), u, jnp.int32), c31)
          for k in range(BPW // 16):
            x16 = plsc.load_gather(rows_v, [m_ks[k], d])
            plsc.store_scatter(tiles_v, [d, m_ks[k]], x16)
        return carry
      lax.fori_loop(0, D // 4, cbody, 0)

    fire_g(rows_v0, sg0, 0)
    fire_g(rows_v1, sg1, 1)

    def body(g, carry):
      l0 = 2 * g

      drain_g(rows_v0, sg0, l0)
      @pl.when(g > 0)
      def _():
        drain_w(tiles_v0, sw0, l0 - 2)
      transpose(rows_v0, tiles_v0)
      fire_w(tiles_v0, sw0, l0)
      @pl.when(g < n2 - 1)
      def _():
        fire_g(rows_v0, sg0, l0 + 2)

      drain_g(rows_v1, sg1, l0 + 1)
      @pl.when(g > 0)
      def _():
        drain_w(tiles_v1, sw1, l0 - 1)
      transpose(rows_v1, tiles_v1)
      fire_w(tiles_v1, sw1, l0 + 1)
      @pl.when(g < n2 - 1)
      def _():
        fire_g(rows_v1, sg1, l0 + 3)
      return carry

    lax.fori_loop(0, n2, body, 0)

    drain_w(tiles_v0, sw0, SEQ - 2)
    drain_w(tiles_v1, sw1, SEQ - 1)

  return gather_kernel


_gather = _make_kernel()


def kernel(x, hidden, emb):
  del hidden
  xt = jnp.transpose(x.astype(jnp.int32))
  out5 = _gather(emb, xt)
  out6 = jnp.transpose(out5, (0, 2, 4, 1, 3))
  return out6.reshape(SEQ, BATCH, D)
